# Initial kernel scaffold; baseline (speedup 1.0000x reference)
import jax
import jax.numpy as jnp
from jax.experimental import pallas as pl


def kernel(edge_index, edge_weight, embeds):
    raise NotImplementedError("write your pallas kernel here")


# speedup vs baseline: 2.8578x; 2.8578x over previous
import functools

import jax
import jax.numpy as jnp
from jax import lax
from jax.experimental import pallas as pl
from jax.experimental.pallas import tpu as pltpu
from jax.experimental.pallas import tpu_sc as plsc

N = 10000
E = 320000
D = 128
H = D // 2
LN_EPS = 1e-5

NUM_CORES = 2
NUM_SUBCORES = 16
CHUNK = 128
NCHUNKS = E // CHUNK
ROWS_MAIN = 624
TAIL_BASE = NUM_SUBCORES * ROWS_MAIN
TAIL_ROWS = N - TAIL_BASE


_LN_BLK = 1000


def _ln_body(x_ref, o_ref):
    x = x_ref[...]
    m = jnp.mean(x, axis=-1, keepdims=True)
    d = x - m
    v = jnp.mean(d * d, axis=-1, keepdims=True)
    y = d * lax.rsqrt(v + LN_EPS)
    o_ref[0] = y[:, :H]
    o_ref[1] = y[:, H:]


def _layernorm_split(embeds):
    grid = N // _LN_BLK
    return pl.pallas_call(
        _ln_body,
        grid=(grid,),
        in_specs=[pl.BlockSpec((_LN_BLK, D), lambda i: (i, 0))],
        out_specs=pl.BlockSpec((2, _LN_BLK, H), lambda i: (0, i, 0)),
        out_shape=jax.ShapeDtypeStruct((2, N, H), jnp.float32),
    )(embeds)


def _edge_pass(ei_hbm, ew_hbm, x_hbm, src_v, dst_v, w_v, rows_v, acc_sh,
               s, row_off):
    nchunks = (NCHUNKS - s + NUM_SUBCORES - 1) // NUM_SUBCORES

    def chunk_body(k, carry):
        cidx = s + NUM_SUBCORES * k
        off = CHUNK * cidx
        pltpu.sync_copy(ei_hbm.at[1, pl.ds(off, CHUNK)], src_v)
        pltpu.sync_copy(ei_hbm.at[0, pl.ds(off, CHUNK)], dst_v)
        pltpu.sync_copy(ew_hbm.at[pl.ds(off, CHUNK)], w_v.at[pl.ds(0, CHUNK)])
        for j in range(CHUNK // 16):
            sl = pl.ds(16 * j, 16)
            src_v[sl] = src_v[sl] + row_off
        pltpu.sync_copy(x_hbm.at[src_v], rows_v)

        def scale(e, c2):
            w = w_v[pl.ds(e, 16)][0]
            for j in range(H // 16):
                sl = pl.ds(16 * j, 16)
                rows_v[e, sl] = rows_v[e, sl] * w
            return c2

        lax.fori_loop(0, CHUNK, scale, 0)
        pltpu.sync_copy(rows_v, acc_sh.at[dst_v], add=True)
        return carry

    lax.fori_loop(0, nchunks, chunk_body, 0)


def _gnn_body(ei_hbm, ew_hbm, x0_hbm, out_hbm, x1_hbm, src_v, dst_v, w_v,
              rows_v, acc_sh):
    c = lax.axis_index("c")
    s = lax.axis_index("s")
    base = s * ROWS_MAIN
    row_off = c * N
    is_tail = s == NUM_SUBCORES - 1

    def zrow(i, carry):
        for j in range(H // 16):
            rows_v[i, pl.ds(16 * j, 16)] = jnp.zeros((16,), jnp.float32)
        return carry

    lax.fori_loop(0, CHUNK, zrow, 0)
    nfull = ROWS_MAIN // CHUNK
    rem = ROWS_MAIN - nfull * CHUNK
    for k in range(nfull):
        pltpu.sync_copy(rows_v, acc_sh.at[pl.ds(base + CHUNK * k, CHUNK)])
    if rem:
        pltpu.sync_copy(rows_v.at[pl.ds(0, rem)],
                        acc_sh.at[pl.ds(base + CHUNK * nfull, rem)])

    @pl.when(is_tail)
    def _():
        pltpu.sync_copy(rows_v.at[pl.ds(0, TAIL_ROWS)],
                        acc_sh.at[pl.ds(TAIL_BASE, TAIL_ROWS)])

    plsc.subcore_barrier()

    _edge_pass(ei_hbm, ew_hbm, x0_hbm, src_v, dst_v, w_v, rows_v, acc_sh,
               s, row_off)
    plsc.subcore_barrier()

    def _publish_x1(off, n):
        pltpu.sync_copy(acc_sh.at[pl.ds(off, n)], rows_v.at[pl.ds(0, n)])
        pltpu.sync_copy(rows_v.at[pl.ds(0, n)],
                        x1_hbm.at[pl.ds(row_off + off, n)])

    for k in range(nfull):
        _publish_x1(base + CHUNK * k, CHUNK)
    if rem:
        _publish_x1(base + CHUNK * nfull, rem)

    @pl.when(is_tail)
    def _():
        _publish_x1(TAIL_BASE, TAIL_ROWS)

    plsc.subcore_barrier()

    _edge_pass(ei_hbm, ew_hbm, x1_hbm, src_v, dst_v, w_v, rows_v, acc_sh,
               s, row_off)
    plsc.subcore_barrier()

    pltpu.sync_copy(acc_sh.at[pl.ds(base, ROWS_MAIN)],
                    out_hbm.at[c, pl.ds(base, ROWS_MAIN)])

    @pl.when(is_tail)
    def _():
        pltpu.sync_copy(acc_sh.at[pl.ds(TAIL_BASE, TAIL_ROWS)],
                        out_hbm.at[c, pl.ds(TAIL_BASE, TAIL_ROWS)])


def _gnn(edge_index, edge_weight, x0):
    mesh = plsc.VectorSubcoreMesh(core_axis_name="c", subcore_axis_name="s")
    out, _ = pl.kernel(
        _gnn_body,
        out_type=(
            jax.ShapeDtypeStruct((2, N, H), jnp.float32),
            jax.ShapeDtypeStruct((2 * N, H), jnp.float32),
        ),
        mesh=mesh,
        scratch_types=[
            pltpu.VMEM((CHUNK,), jnp.int32),
            pltpu.VMEM((CHUNK,), jnp.int32),
            pltpu.VMEM((CHUNK + 16,), jnp.float32),
            pltpu.VMEM((CHUNK, H), jnp.float32),
            pltpu.VMEM_SHARED((N, H), jnp.float32),
        ],
        compiler_params=pltpu.CompilerParams(use_tc_tiling_on_sc=False),
    )(edge_index, edge_weight, x0)
    return out


@jax.jit
def kernel(edge_index, edge_weight, embeds):
    x0 = _layernorm_split(embeds)
    x0f = x0.reshape(2 * N, H)
    out = _gnn(edge_index, edge_weight, x0f)
    return out.transpose(1, 0, 2).reshape(N, D)


# speedup vs baseline: 8.6293x; 3.0195x over previous
import jax
import jax.numpy as jnp
from jax import lax
from jax.experimental import pallas as pl
from jax.experimental.pallas import tpu as pltpu
from jax.experimental.pallas import tpu_sc as plsc

N = 10000
E = 320000
D = 128
H = D // 2
LN_EPS = 1e-5

NUM_SUBCORES = 16
CHUNK = 128
NCHUNKS = E // CHUNK
MAIN_CHUNKS = NCHUNKS // NUM_SUBCORES
EXTRA_BASE = MAIN_CHUNKS * NUM_SUBCORES
NPAIRS = MAIN_CHUNKS // 2
ROWS_MAIN = 624
TAIL_BASE = NUM_SUBCORES * ROWS_MAIN
TAIL_ROWS = N - TAIL_BASE


_LN_BLK = 1000


def _ln_body(x_ref, o_ref):
    x = x_ref[...]
    m = jnp.mean(x, axis=-1, keepdims=True)
    d = x - m
    v = jnp.mean(d * d, axis=-1, keepdims=True)
    y = d * lax.rsqrt(v + LN_EPS)
    o_ref[0] = y[:, :H]
    o_ref[1] = y[:, H:]


def _layernorm_split(embeds):
    grid = N // _LN_BLK
    return pl.pallas_call(
        _ln_body,
        grid=(grid,),
        in_specs=[pl.BlockSpec((_LN_BLK, D), lambda i: (i, 0))],
        out_specs=pl.BlockSpec((2, _LN_BLK, H), lambda i: (0, i, 0)),
        out_shape=jax.ShapeDtypeStruct((2, N, H), jnp.float32),
    )(embeds)


def _scale_chunk(rows_ref, w_st, k):
    def g_body(g, carry):
        wv = w_st[k, pl.ds(16 * g, 16)]
        for i in range(16):
            w = wv[i]
            for j in range(H // 16):
                sl = pl.ds(16 * j, 16)
                rows_ref[16 * g + i, sl] = rows_ref[16 * g + i, sl] * w
        return carry

    lax.fori_loop(0, CHUNK // 16, g_body, 0)


def _edge_pass(x_hbm, src_st, dst_st, w_st, rows0, rows1, acc_sh,
               gs0, gs1, ss0, ss1, has_extra):
    def gather(k, rows, sem):
        return pltpu.async_copy(x_hbm.at[src_st.at[k]], rows, sem)

    def gather_wait(k, rows, sem):
        pltpu.make_async_copy(x_hbm.at[src_st.at[k]], rows, sem).wait()

    def scatter(k, rows, sem):
        return pltpu.async_copy(rows, acc_sh.at[dst_st.at[k]], sem, add=True)

    def scatter_wait(k, rows, sem):
        pltpu.make_async_copy(rows, acc_sh.at[dst_st.at[k]], sem).wait()

    gather(0, rows0, gs0)

    def pair_body(p, carry):
        k0 = 2 * p
        k1 = k0 + 1
        gather_wait(k0, rows0, gs0)

        @pl.when(p > 0)
        def _():
            scatter_wait(k1 - 2, rows1, ss1)

        gather(k1, rows1, gs1)
        _scale_chunk(rows0, w_st, k0)
        scatter(k0, rows0, ss0)
        gather_wait(k1, rows1, gs1)

        @pl.when(p + 1 < NPAIRS)
        def _():
            scatter_wait(k0, rows0, ss0)
            gather(k0 + 2, rows0, gs0)

        _scale_chunk(rows1, w_st, k1)
        scatter(k1, rows1, ss1)
        return carry

    lax.fori_loop(0, NPAIRS, pair_body, 0)
    scatter_wait(0, rows0, ss0)
    scatter_wait(0, rows1, ss1)

    @pl.when(has_extra)
    def _():
        ke = MAIN_CHUNKS
        pltpu.sync_copy(x_hbm.at[src_st.at[ke]], rows0)
        _scale_chunk(rows0, w_st, ke)
        pltpu.sync_copy(rows0, acc_sh.at[dst_st.at[ke]], add=True)


def _gnn_body(src2d, dst2d, w2d, x0_hbm, out_hbm, x1_hbm,
              src_st, dst_st, w_st, rows0, rows1, acc_sh,
              gs0, gs1, ss0, ss1):
    c = lax.axis_index("c")
    s = lax.axis_index("s")
    base = s * ROWS_MAIN
    row_off = c * N
    is_tail = s == NUM_SUBCORES - 1
    has_extra = s < NCHUNKS - EXTRA_BASE

    cbase = s * MAIN_CHUNKS
    pltpu.sync_copy(src2d.at[pl.ds(cbase, MAIN_CHUNKS)],
                    src_st.at[pl.ds(0, MAIN_CHUNKS)])
    pltpu.sync_copy(dst2d.at[pl.ds(cbase, MAIN_CHUNKS)],
                    dst_st.at[pl.ds(0, MAIN_CHUNKS)])
    pltpu.sync_copy(w2d.at[pl.ds(cbase, MAIN_CHUNKS)],
                    w_st.at[pl.ds(0, MAIN_CHUNKS)])

    @pl.when(has_extra)
    def _():
        eb = EXTRA_BASE + s
        pltpu.sync_copy(src2d.at[pl.ds(eb, 1)],
                        src_st.at[pl.ds(MAIN_CHUNKS, 1)])
        pltpu.sync_copy(dst2d.at[pl.ds(eb, 1)],
                        dst_st.at[pl.ds(MAIN_CHUNKS, 1)])
        pltpu.sync_copy(w2d.at[pl.ds(eb, 1)],
                        w_st.at[pl.ds(MAIN_CHUNKS, 1)])

    def shift_body(i, carry):
        for j in range(CHUNK // 16):
            sl = pl.ds(16 * j, 16)
            src_st[i, sl] = src_st[i, sl] + row_off
        return carry

    nst = MAIN_CHUNKS + 1
    lax.fori_loop(0, nst, shift_body, 0)

    def zrow(i, carry):
        for j in range(H // 16):
            rows0[i, pl.ds(16 * j, 16)] = jnp.zeros((16,), jnp.float32)
        return carry

    lax.fori_loop(0, CHUNK, zrow, 0)
    nfull = ROWS_MAIN // CHUNK
    rem = ROWS_MAIN - nfull * CHUNK
    for k in range(nfull):
        pltpu.sync_copy(rows0, acc_sh.at[pl.ds(base + CHUNK * k, CHUNK)])
    if rem:
        pltpu.sync_copy(rows0.at[pl.ds(0, rem)],
                        acc_sh.at[pl.ds(base + CHUNK * nfull, rem)])

    @pl.when(is_tail)
    def _():
        pltpu.sync_copy(rows0.at[pl.ds(0, TAIL_ROWS)],
                        acc_sh.at[pl.ds(TAIL_BASE, TAIL_ROWS)])

    plsc.subcore_barrier()

    _edge_pass(x0_hbm, src_st, dst_st, w_st, rows0, rows1, acc_sh,
               gs0, gs1, ss0, ss1, has_extra)
    plsc.subcore_barrier()

    def _publish_x1(off, n):
        pltpu.sync_copy(acc_sh.at[pl.ds(off, n)], rows0.at[pl.ds(0, n)])
        pltpu.sync_copy(rows0.at[pl.ds(0, n)],
                        x1_hbm.at[pl.ds(row_off + off, n)])

    for k in range(nfull):
        _publish_x1(base + CHUNK * k, CHUNK)
    if rem:
        _publish_x1(base + CHUNK * nfull, rem)

    @pl.when(is_tail)
    def _():
        _publish_x1(TAIL_BASE, TAIL_ROWS)

    plsc.subcore_barrier()

    _edge_pass(x1_hbm, src_st, dst_st, w_st, rows0, rows1, acc_sh,
               gs0, gs1, ss0, ss1, has_extra)
    plsc.subcore_barrier()

    pltpu.sync_copy(acc_sh.at[pl.ds(base, ROWS_MAIN)],
                    out_hbm.at[c, pl.ds(base, ROWS_MAIN)])

    @pl.when(is_tail)
    def _():
        pltpu.sync_copy(acc_sh.at[pl.ds(TAIL_BASE, TAIL_ROWS)],
                        out_hbm.at[c, pl.ds(TAIL_BASE, TAIL_ROWS)])


def _gnn(src2d, dst2d, w2d, x0f):
    mesh = plsc.VectorSubcoreMesh(core_axis_name="c", subcore_axis_name="s")
    out, _ = pl.kernel(
        _gnn_body,
        out_type=(
            jax.ShapeDtypeStruct((2, N, H), jnp.float32),
            jax.ShapeDtypeStruct((2 * N, H), jnp.float32),
        ),
        mesh=mesh,
        scratch_types=[
            pltpu.VMEM((MAIN_CHUNKS + 1, CHUNK), jnp.int32),
            pltpu.VMEM((MAIN_CHUNKS + 1, CHUNK), jnp.int32),
            pltpu.VMEM((MAIN_CHUNKS + 1, CHUNK), jnp.float32),
            pltpu.VMEM((CHUNK, H), jnp.float32),
            pltpu.VMEM((CHUNK, H), jnp.float32),
            pltpu.VMEM_SHARED((N, H), jnp.float32),
            pltpu.SemaphoreType.DMA,
            pltpu.SemaphoreType.DMA,
            pltpu.SemaphoreType.DMA,
            pltpu.SemaphoreType.DMA,
        ],
        compiler_params=pltpu.CompilerParams(use_tc_tiling_on_sc=False),
    )(src2d, dst2d, w2d, x0f)
    return out


@jax.jit
def kernel(edge_index, edge_weight, embeds):
    x0 = _layernorm_split(embeds)
    x0f = x0.reshape(2 * N, H)
    src2d = edge_index[1].reshape(NCHUNKS, CHUNK)
    dst2d = edge_index[0].reshape(NCHUNKS, CHUNK)
    w2d = edge_weight.reshape(NCHUNKS, CHUNK)
    out = _gnn(src2d, dst2d, w2d, x0f)
    return out.transpose(1, 0, 2).reshape(N, D)


# speedup vs baseline: 8.7450x; 1.0134x over previous
import jax
import jax.numpy as jnp
from jax import lax
from jax.experimental import pallas as pl
from jax.experimental.pallas import tpu as pltpu
from jax.experimental.pallas import tpu_sc as plsc

N = 10000
E = 320000
D = 128
H = D // 2
LN_EPS = 1e-5

NUM_SUBCORES = 16
CHUNK = 128
NCHUNKS = E // CHUNK
MAIN_CHUNKS = NCHUNKS // NUM_SUBCORES
EXTRA_BASE = MAIN_CHUNKS * NUM_SUBCORES
NPAIRS = MAIN_CHUNKS // 2
ROWS_MAIN = 624
TAIL_BASE = NUM_SUBCORES * ROWS_MAIN
TAIL_ROWS = N - TAIL_BASE


_LN_BLK = 1000


def _ln_body(x_ref, o_ref):
    x = x_ref[...]
    m = jnp.mean(x, axis=-1, keepdims=True)
    d = x - m
    v = jnp.mean(d * d, axis=-1, keepdims=True)
    y = d * lax.rsqrt(v + LN_EPS)
    o_ref[0] = y[:, :H]
    o_ref[1] = y[:, H:]


def _layernorm_split(embeds):
    grid = N // _LN_BLK
    return pl.pallas_call(
        _ln_body,
        grid=(grid,),
        in_specs=[pl.BlockSpec((_LN_BLK, D), lambda i: (i, 0))],
        out_specs=pl.BlockSpec((2, _LN_BLK, H), lambda i: (0, i, 0)),
        out_shape=jax.ShapeDtypeStruct((2, N, H), jnp.float32),
    )(embeds)


def _scale_chunk(rows_ref, w_st, k):
    def g_body(g, carry):
        wv = w_st[k, pl.ds(16 * g, 16)]
        for i in range(16):
            w = wv[i]
            for j in range(H // 16):
                sl = pl.ds(16 * j, 16)
                rows_ref[16 * g + i, sl] = rows_ref[16 * g + i, sl] * w
        return carry

    lax.fori_loop(0, CHUNK // 16, g_body, 0)


def _edge_pass(x_hbm, src_st, dst_st, w_st, bufs, acc_sh,
               gsems, ssems, has_extra):
    nbuf = len(bufs)

    def gather(k, b):
        pltpu.async_copy(x_hbm.at[src_st.at[k]], bufs[b], gsems[b])

    def gather_wait(k, b):
        pltpu.make_async_copy(x_hbm.at[src_st.at[k]], bufs[b],
                              gsems[b]).wait()

    def scatter(k, b):
        pltpu.async_copy(bufs[b], acc_sh.at[dst_st.at[k]], ssems[b],
                         add=True)

    def scatter_wait(k, b):
        pltpu.make_async_copy(bufs[b], acc_sh.at[dst_st.at[k]],
                              ssems[b]).wait()

    gather(0, 0)
    gather(1, 1)

    def group_body(g, carry):
        for b in range(nbuf):
            k = nbuf * g + b
            gather_wait(k, b)
            b2 = (b + 2) % nbuf

            @pl.when(k + 2 < MAIN_CHUNKS)
            def _():
                @pl.when(k + 2 >= nbuf)
                def _():
                    scatter_wait(k + 2 - nbuf, b2)

                gather(k + 2, b2)

            _scale_chunk(bufs[b], w_st, k)
            scatter(k, b)
        return carry

    lax.fori_loop(0, MAIN_CHUNKS // nbuf, group_body, 0)
    for b in range(nbuf):
        scatter_wait(0, b)

    @pl.when(has_extra)
    def _():
        ke = MAIN_CHUNKS
        pltpu.sync_copy(x_hbm.at[src_st.at[ke]], bufs[0])
        _scale_chunk(bufs[0], w_st, ke)
        pltpu.sync_copy(bufs[0], acc_sh.at[dst_st.at[ke]], add=True)


def _gnn_body(src2d, dst2d, w2d, x0_hbm, out_hbm, x1_hbm,
              src_st, dst_st, w_st, rows0, rows1, rows2, acc_sh,
              gs0, gs1, gs2, ss0, ss1, ss2):
    bufs = (rows0, rows1, rows2)
    gsems = (gs0, gs1, gs2)
    ssems = (ss0, ss1, ss2)
    c = lax.axis_index("c")
    s = lax.axis_index("s")
    base = s * ROWS_MAIN
    row_off = c * N
    is_tail = s == NUM_SUBCORES - 1
    has_extra = s < NCHUNKS - EXTRA_BASE

    cbase = s * MAIN_CHUNKS
    pltpu.sync_copy(src2d.at[pl.ds(cbase, MAIN_CHUNKS)],
                    src_st.at[pl.ds(0, MAIN_CHUNKS)])
    pltpu.sync_copy(dst2d.at[pl.ds(cbase, MAIN_CHUNKS)],
                    dst_st.at[pl.ds(0, MAIN_CHUNKS)])
    pltpu.sync_copy(w2d.at[pl.ds(cbase, MAIN_CHUNKS)],
                    w_st.at[pl.ds(0, MAIN_CHUNKS)])

    @pl.when(has_extra)
    def _():
        eb = EXTRA_BASE + s
        pltpu.sync_copy(src2d.at[pl.ds(eb, 1)],
                        src_st.at[pl.ds(MAIN_CHUNKS, 1)])
        pltpu.sync_copy(dst2d.at[pl.ds(eb, 1)],
                        dst_st.at[pl.ds(MAIN_CHUNKS, 1)])
        pltpu.sync_copy(w2d.at[pl.ds(eb, 1)],
                        w_st.at[pl.ds(MAIN_CHUNKS, 1)])

    def shift_body(i, carry):
        for j in range(CHUNK // 16):
            sl = pl.ds(16 * j, 16)
            src_st[i, sl] = src_st[i, sl] + row_off
        return carry

    nst = MAIN_CHUNKS + 1
    lax.fori_loop(0, nst, shift_body, 0)

    def zrow(i, carry):
        for j in range(H // 16):
            rows0[i, pl.ds(16 * j, 16)] = jnp.zeros((16,), jnp.float32)
        return carry

    lax.fori_loop(0, CHUNK, zrow, 0)
    nfull = ROWS_MAIN // CHUNK
    rem = ROWS_MAIN - nfull * CHUNK
    for k in range(nfull):
        pltpu.sync_copy(rows0, acc_sh.at[pl.ds(base + CHUNK * k, CHUNK)])
    if rem:
        pltpu.sync_copy(rows0.at[pl.ds(0, rem)],
                        acc_sh.at[pl.ds(base + CHUNK * nfull, rem)])

    @pl.when(is_tail)
    def _():
        pltpu.sync_copy(rows0.at[pl.ds(0, TAIL_ROWS)],
                        acc_sh.at[pl.ds(TAIL_BASE, TAIL_ROWS)])

    plsc.subcore_barrier()

    _edge_pass(x0_hbm, src_st, dst_st, w_st, bufs, acc_sh,
               gsems, ssems, has_extra)
    plsc.subcore_barrier()

    def _publish_x1(off, n):
        pltpu.sync_copy(acc_sh.at[pl.ds(off, n)], rows0.at[pl.ds(0, n)])
        pltpu.sync_copy(rows0.at[pl.ds(0, n)],
                        x1_hbm.at[pl.ds(row_off + off, n)])

    for k in range(nfull):
        _publish_x1(base + CHUNK * k, CHUNK)
    if rem:
        _publish_x1(base + CHUNK * nfull, rem)

    @pl.when(is_tail)
    def _():
        _publish_x1(TAIL_BASE, TAIL_ROWS)

    plsc.subcore_barrier()

    _edge_pass(x1_hbm, src_st, dst_st, w_st, bufs, acc_sh,
               gsems, ssems, has_extra)
    plsc.subcore_barrier()

    pltpu.sync_copy(acc_sh.at[pl.ds(base, ROWS_MAIN)],
                    out_hbm.at[c, pl.ds(base, ROWS_MAIN)])

    @pl.when(is_tail)
    def _():
        pltpu.sync_copy(acc_sh.at[pl.ds(TAIL_BASE, TAIL_ROWS)],
                        out_hbm.at[c, pl.ds(TAIL_BASE, TAIL_ROWS)])


def _gnn(src2d, dst2d, w2d, x0f):
    mesh = plsc.VectorSubcoreMesh(core_axis_name="c", subcore_axis_name="s")
    out, _ = pl.kernel(
        _gnn_body,
        out_type=(
            jax.ShapeDtypeStruct((2, N, H), jnp.float32),
            jax.ShapeDtypeStruct((2 * N, H), jnp.float32),
        ),
        mesh=mesh,
        scratch_types=[
            pltpu.VMEM((MAIN_CHUNKS + 1, CHUNK), jnp.int32),
            pltpu.VMEM((MAIN_CHUNKS + 1, CHUNK), jnp.int32),
            pltpu.VMEM((MAIN_CHUNKS + 1, CHUNK), jnp.float32),
            pltpu.VMEM((CHUNK, H), jnp.float32),
            pltpu.VMEM((CHUNK, H), jnp.float32),
            pltpu.VMEM((CHUNK, H), jnp.float32),
            pltpu.VMEM_SHARED((N, H), jnp.float32),
            pltpu.SemaphoreType.DMA,
            pltpu.SemaphoreType.DMA,
            pltpu.SemaphoreType.DMA,
            pltpu.SemaphoreType.DMA,
            pltpu.SemaphoreType.DMA,
            pltpu.SemaphoreType.DMA,
        ],
        compiler_params=pltpu.CompilerParams(use_tc_tiling_on_sc=False),
    )(src2d, dst2d, w2d, x0f)
    return out


@jax.jit
def kernel(edge_index, edge_weight, embeds):
    x0 = _layernorm_split(embeds)
    x0f = x0.reshape(2 * N, H)
    src2d = edge_index[1].reshape(NCHUNKS, CHUNK)
    dst2d = edge_index[0].reshape(NCHUNKS, CHUNK)
    w2d = edge_weight.reshape(NCHUNKS, CHUNK)
    out = _gnn(src2d, dst2d, w2d, x0f)
    return out.transpose(1, 0, 2).reshape(N, D)
